# baseline (speedup 1.0000x reference)
import functools

import jax
import jax.numpy as jnp
from jax import lax
from jax.experimental import pallas as pl
from jax.experimental.pallas import tpu as pltpu
from jax.experimental.pallas import tpu_sc as plsc

N = 10000
E = 320000
D = 128
H = 128
C = 40
NCOMM = 100

NP = 10240
NCORE = 2
NSUB = 16
NWORK = NCORE * NSUB
CHUNK = 128
NCHS = 160
HALF = NCHS // 2
EPAD = NSUB * NCHS * CHUNK
ROWS_PER_SUB = NP // NSUB
DUMP_ROW = N + 64
WH1 = D // 2
WH2 = 32

_f32 = jnp.float32
_i32 = jnp.int32


def _h0_body(x_ref, comm_ref, wlin_ref, blin_ref):
    x = x_ref[...]
    comm = comm_ref[...]
    ids = lax.broadcasted_iota(_i32, (N, NCOMM), 1)
    onehot = (comm == ids).astype(_f32)
    csum = lax.dot_general(onehot, x, (((0,), (0,)), ((), ())),
                           preferred_element_type=_f32)
    cnt = jnp.sum(onehot, axis=0)[:, None]
    cmean = csum / jnp.maximum(cnt, 1.0)
    xc = jnp.dot(onehot, cmean, preferred_element_type=_f32)
    wlin = wlin_ref[...]
    h0 = x @ wlin[0:D] + xc @ wlin[D:2 * D] + blin_ref[...]
    return jnp.maximum(h0, 0.0)


def _deg_body(dst_hbm, out_hbm, idx_v, acc_v):
    cid = lax.axis_index("c")
    sid = lax.axis_index("s")
    wid = sid * NCORE + cid
    pltpu.sync_copy(dst_hbm.at[sid, pl.ds(cid * HALF, HALF)], idx_v)

    def _zero(i, _):
        acc_v[pl.ds(i * 16, 16)] = jnp.zeros((16,), _f32)
        return 0
    lax.fori_loop(0, NP // 16, _zero, 0)

    ones16 = jnp.full((16,), 1.0, _f32)

    def _edges(c, _):
        def _sub(j, __):
            idx = idx_v[c, pl.ds(j * 16, 16)]
            plsc.addupdate_scatter(acc_v, [idx], ones16)
            return 0
        lax.fori_loop(0, 8, _sub, 0)
        return 0
    lax.fori_loop(0, HALF, _edges, 0)
    pltpu.sync_copy(acc_v, out_hbm.at[wid])


_deg_call = functools.partial(
    pl.kernel,
    out_type=jax.ShapeDtypeStruct((NWORK, NP), _f32),
    mesh=plsc.VectorSubcoreMesh(core_axis_name="c", subcore_axis_name="s"),
    compiler_params=pltpu.CompilerParams(needs_layout_passes=False),
    scratch_types=[
        pltpu.VMEM((HALF, CHUNK), _i32),
        pltpu.VMEM((NP,), _f32),
    ],
)(_deg_body)


def _prep_body(degp_ref, x_ref, comm_ref, wlin_ref, blin_ref,
               dinv_ref, dinv1_ref, g_ref):
    h0 = _h0_body(x_ref, comm_ref, wlin_ref, blin_ref)
    deg = jnp.sum(degp_ref[...], axis=0) + 1.0
    dinv1 = lax.rsqrt(deg)
    dinv = dinv1[:, None]
    dinv_ref[...] = dinv
    dinv1_ref[...] = dinv1
    g = h0 * dinv[0:N]
    zpad = jnp.zeros((NP - N, WH1), _f32)
    g_ref[0, 0:N, :] = g[:, 0:WH1]
    g_ref[0, N:NP, :] = zpad
    g_ref[1, 0:N, :] = g[:, WH1:D]
    g_ref[1, N:NP, :] = zpad


def _prep_call(degp, x, comm2d, W_lin, blin2d):
    return pl.pallas_call(
        _prep_body,
        out_shape=(
            jax.ShapeDtypeStruct((NP, 1), _f32),
            jax.ShapeDtypeStruct((NP,), _f32),
            jax.ShapeDtypeStruct((NCORE, NP, WH1), _f32),
        ),
    )(degp, x, comm2d, W_lin, blin2d)


def _make_agg(wh, final=False):
    stg = 40 if wh > 32 else HALF

    def _body(*refs):
        if final:
            (g_hbm, src_hbm, dst_hbm, dinv_hbm, b2_hbm, out_hbm,
             src_v, dst_v, buf0, buf1, buf2, buf3, dinv_v, b2_v, tab_sh, acc_sh,
             gsem0, gsem1, gsem2, gsem3, ssem0, ssem1, ssem2, ssem3) = refs
        else:
            (g_hbm, src_hbm, dst_hbm, out_hbm,
             src_v, dst_v, buf0, buf1, buf2, buf3, tab_sh, acc_sh,
             gsem0, gsem1, gsem2, gsem3, ssem0, ssem1, ssem2, ssem3) = refs
        bufs = (buf0, buf1, buf2, buf3)
        gsem = (gsem0, gsem1, gsem2, gsem3)
        ssem = (ssem0, ssem1, ssem2, ssem3)
        cid = lax.axis_index("c")
        sid = lax.axis_index("s")
        r0 = sid * ROWS_PER_SUB

        def _zrow(i, _):
            def _zf(f, __):
                bufs[0][i, pl.ds(f * 16, 16)] = jnp.zeros((16,), _f32)
                return 0
            lax.fori_loop(0, wh // 16, _zf, 0)
            return 0
        lax.fori_loop(0, CHUNK, _zrow, 0)
        for k in range(ROWS_PER_SUB // CHUNK):
            pltpu.sync_copy(bufs[0], acc_sh.at[pl.ds(r0 + k * CHUNK, CHUNK), :])
        pltpu.sync_copy(g_hbm.at[cid, pl.ds(r0, ROWS_PER_SUB), :],
                        tab_sh.at[pl.ds(r0, ROWS_PER_SUB), :])
        plsc.subcore_barrier()

        def _gwait(b):
            pltpu.make_async_copy(tab_sh.at[src_v.at[0]], bufs[b], gsem[b]).wait()

        def _swait(b):
            pltpu.make_async_copy(bufs[b], acc_sh.at[dst_v.at[0]], ssem[b]).wait()

        for h in range(NCHS // stg):
            pltpu.sync_copy(src_hbm.at[sid, pl.ds(h * stg, stg)], src_v)
            pltpu.sync_copy(dst_hbm.at[sid, pl.ds(h * stg, stg)], dst_v)
            for b in range(3):
                pltpu.async_copy(tab_sh.at[src_v.at[b]], bufs[b], gsem[b])
            _gwait(0)
            pltpu.async_copy(bufs[0], acc_sh.at[dst_v.at[0]], ssem[0], add=True)
            pltpu.async_copy(tab_sh.at[src_v.at[3]], bufs[3], gsem[3])

            def _step(s, _):
                for u in range(4):
                    j = s * 4 + u + 1
                    b = (u + 1) % 4
                    _gwait(b)
                    pltpu.async_copy(bufs[b], acc_sh.at[dst_v.at[j]], ssem[b], add=True)
                    _swait((b + 3) % 4)
                    pltpu.async_copy(tab_sh.at[src_v.at[j + 3]], bufs[(b + 3) % 4],
                                     gsem[(b + 3) % 4])
                return 0
            lax.fori_loop(0, (stg - 4) // 4, _step, 0)

            for j in range(stg - 3, stg):
                b = j % 4
                _gwait(b)
                pltpu.async_copy(bufs[b], acc_sh.at[dst_v.at[j]], ssem[b], add=True)
                _swait((b + 3) % 4)
            _swait((stg - 1) % 4)

        plsc.subcore_barrier()
        if not final:
            pltpu.sync_copy(acc_sh.at[pl.ds(r0, ROWS_PER_SUB), :],
                            out_hbm.at[cid, pl.ds(r0, ROWS_PER_SUB), :])
        else:
            pltpu.sync_copy(dinv_hbm.at[pl.ds(r0, ROWS_PER_SUB)], dinv_v)
            pltpu.sync_copy(b2_hbm.at[cid], b2_v)
            bias = [b2_v[pl.ds(f * 16, 16)] for f in range(wh // 16)]
            for k in range(ROWS_PER_SUB // CHUNK):
                pltpu.sync_copy(acc_sh.at[pl.ds(r0 + k * CHUNK, CHUNK), :], buf0)
                pltpu.sync_copy(tab_sh.at[pl.ds(r0 + k * CHUNK, CHUNK), :], buf1)

                def _row(i, _, k=k):
                    dv = plsc.load_gather(
                        dinv_v, [jnp.broadcast_to(k * CHUNK + i, (16,))])
                    for f in range(wh // 16):
                        cs = pl.ds(f * 16, 16)
                        buf2[i, cs] = (buf0[i, cs] + buf1[i, cs]) * dv + bias[f]
                    return 0
                lax.fori_loop(0, CHUNK, _row, 0)

                rowg = r0 + k * CHUNK

                @pl.when(jnp.logical_and(rowg + CHUNK <= N, cid == 0))
                def _():
                    pltpu.sync_copy(buf2, out_hbm.at[pl.ds(rowg, CHUNK), pl.ds(0, WH2)])

                @pl.when(jnp.logical_and(rowg + CHUNK <= N, cid == 1))
                def _():
                    pltpu.sync_copy(buf2.at[:, pl.ds(0, C - WH2)],
                                    out_hbm.at[pl.ds(rowg, CHUNK), pl.ds(WH2, C - WH2)])

                @pl.when(jnp.logical_and(rowg == (N // CHUNK) * CHUNK, cid == 0))
                def _():
                    pltpu.sync_copy(buf2.at[pl.ds(0, N % CHUNK), :],
                                    out_hbm.at[pl.ds(rowg, N % CHUNK), pl.ds(0, WH2)])

                @pl.when(jnp.logical_and(rowg == (N // CHUNK) * CHUNK, cid == 1))
                def _():
                    pltpu.sync_copy(buf2.at[pl.ds(0, N % CHUNK), pl.ds(0, C - WH2)],
                                    out_hbm.at[pl.ds(rowg, N % CHUNK), pl.ds(WH2, C - WH2)])

    if final:
        out_type = jax.ShapeDtypeStruct((N, C), _f32)
        extra_scr = [pltpu.VMEM((ROWS_PER_SUB,), _f32), pltpu.VMEM((wh,), _f32)]
    else:
        out_type = jax.ShapeDtypeStruct((NCORE, NP, wh), _f32)
        extra_scr = []
    return functools.partial(
        pl.kernel,
        out_type=out_type,
        mesh=plsc.VectorSubcoreMesh(core_axis_name="c", subcore_axis_name="s"),
        compiler_params=pltpu.CompilerParams(
            needs_layout_passes=False,
            use_tc_tiling_on_sc=False,
        ),
        scratch_types=(
            [pltpu.VMEM((stg, CHUNK), _i32)] * 2
            + [pltpu.VMEM((CHUNK, wh), _f32)] * 4
            + extra_scr
            + [pltpu.VMEM_SHARED((NP, wh), _f32)] * 2
            + [pltpu.SemaphoreType.DMA] * 8
        ),
    )(_body)


_agg_d = _make_agg(WH1)
_agg_w = _make_agg(WH2, final=True)


def _mid_body(agg_ref, g_ref, dinv_ref, w1_ref, b1_ref, w2_ref, q_ref):
    s = jnp.concatenate([agg_ref[0], agg_ref[1]], axis=-1)
    gg = jnp.concatenate([g_ref[0], g_ref[1]], axis=-1)
    dinv = dinv_ref[...]
    agg1 = dinv * (s + gg)
    h1 = jnp.maximum(agg1 @ w1_ref[...] + b1_ref[...], 0.0)
    q = dinv * (h1 @ w2_ref[...])
    q_ref[0, :, :] = q[:, 0:WH2]
    q_ref[1, :, :] = q[:, WH2:2 * WH2]


def _mid_call(agg, g, dinv, W1, b1_2d, W2p):
    return pl.pallas_call(
        _mid_body,
        out_shape=jax.ShapeDtypeStruct((NCORE, NP, WH2), _f32),
    )(agg, g, dinv, W1, b1_2d, W2p)


def kernel(x, edge_index, community, W_lin, b_lin, W1, b1, W2, b2):
    src = edge_index[0]
    dst = edge_index[1]
    pad = EPAD - E
    src3 = jnp.concatenate([src, jnp.zeros((pad,), _i32)]).reshape(NSUB, NCHS, CHUNK)
    dst3 = jnp.concatenate([dst, jnp.full((pad,), DUMP_ROW, _i32)]).reshape(NSUB, NCHS, CHUNK)
    W2p = jnp.pad(W2, ((0, 0), (0, 2 * WH2 - C)))
    b2p = jnp.stack([b2[0:WH2], jnp.pad(b2[WH2:C], (0, 2 * WH2 - C))])

    degp = _deg_call(dst3)
    dinv, dinv1, g = _prep_call(degp, x, community.reshape(N, 1), W_lin,
                                b_lin.reshape(1, H))
    aggp = _agg_d(g, src3, dst3)
    q = _mid_call(aggp, g, dinv, W1, b1.reshape(1, H), W2p)
    return _agg_w(q, src3, dst3, dinv1, b2p)


# speedup vs baseline: 1.0092x; 1.0092x over previous
import functools

import jax
import jax.numpy as jnp
from jax import lax
from jax.experimental import pallas as pl
from jax.experimental.pallas import tpu as pltpu
from jax.experimental.pallas import tpu_sc as plsc

N = 10000
E = 320000
D = 128
H = 128
C = 40
NCOMM = 100

NP = 10240
NCORE = 2
NSUB = 16
NWORK = NCORE * NSUB
CHUNK = 128
NCHS = 160
HALF = NCHS // 2
EPAD = NSUB * NCHS * CHUNK
ROWS_PER_SUB = NP // NSUB
DUMP_ROW = N + 64
WH1 = D // 2
WH2 = 32

_f32 = jnp.float32
_i32 = jnp.int32


def _h0_body(x_ref, comm_ref, wlin_ref, blin_ref):
    x = x_ref[...]
    comm = comm_ref[...]
    ids = lax.broadcasted_iota(_i32, (N, NCOMM), 1)
    onehot = (comm == ids).astype(_f32)
    csum = lax.dot_general(onehot, x, (((0,), (0,)), ((), ())),
                           preferred_element_type=_f32)
    cnt = jnp.sum(onehot, axis=0)[:, None]
    cmean = csum / jnp.maximum(cnt, 1.0)
    xc = jnp.dot(onehot, cmean, preferred_element_type=_f32)
    wlin = wlin_ref[...]
    h0 = x @ wlin[0:D] + xc @ wlin[D:2 * D] + blin_ref[...]
    return jnp.maximum(h0, 0.0)


def _deg_body(dst_hbm, out_hbm, idx_v, acc_v):
    cid = lax.axis_index("c")
    sid = lax.axis_index("s")
    wid = sid * NCORE + cid
    pltpu.sync_copy(dst_hbm.at[sid, pl.ds(cid * HALF, HALF)], idx_v)

    def _zero(i, _):
        acc_v[pl.ds(i * 16, 16)] = jnp.zeros((16,), _f32)
        return 0
    lax.fori_loop(0, NP // 16, _zero, 0)

    ones16 = jnp.full((16,), 1.0, _f32)

    def _edges(c, _):
        def _sub(j, __):
            idx = idx_v[c, pl.ds(j * 16, 16)]
            plsc.addupdate_scatter(acc_v, [idx], ones16)
            return 0
        lax.fori_loop(0, 8, _sub, 0)
        return 0
    lax.fori_loop(0, HALF, _edges, 0)
    pltpu.sync_copy(acc_v, out_hbm.at[wid])


_deg_call = functools.partial(
    pl.kernel,
    out_type=jax.ShapeDtypeStruct((NWORK, NP), _f32),
    mesh=plsc.VectorSubcoreMesh(core_axis_name="c", subcore_axis_name="s"),
    compiler_params=pltpu.CompilerParams(needs_layout_passes=False),
    scratch_types=[
        pltpu.VMEM((HALF, CHUNK), _i32),
        pltpu.VMEM((NP,), _f32),
    ],
)(_deg_body)


def _prep_body(degp_ref, x_ref, comm_ref, wlin_ref, blin_ref,
               dinv_ref, dinv1_ref, g_ref):
    h0 = _h0_body(x_ref, comm_ref, wlin_ref, blin_ref)
    deg = jnp.sum(degp_ref[...], axis=0) + 1.0
    dinv1 = lax.rsqrt(deg)
    dinv = dinv1[:, None]
    dinv_ref[...] = dinv
    dinv1_ref[...] = dinv1
    g = h0 * dinv[0:N]
    zpad = jnp.zeros((NP - N, WH1), _f32)
    g_ref[0, 0:N, :] = g[:, 0:WH1]
    g_ref[0, N:NP, :] = zpad
    g_ref[1, 0:N, :] = g[:, WH1:D]
    g_ref[1, N:NP, :] = zpad


def _prep_call(degp, x, comm2d, W_lin, blin2d):
    return pl.pallas_call(
        _prep_body,
        out_shape=(
            jax.ShapeDtypeStruct((NP, 1), _f32),
            jax.ShapeDtypeStruct((NP,), _f32),
            jax.ShapeDtypeStruct((NCORE, NP, WH1), _f32),
        ),
    )(degp, x, comm2d, W_lin, blin2d)


def _make_agg(wh, final=False):
    def _body(*refs):
        if final:
            (g_hbm, src_hbm, dst_hbm, dinv_hbm, b2_hbm, out_hbm,
             src_v, dst_v, buf0, buf1, buf2, dinv_v, b2_v, tab_sh, acc_sh,
             gsem0, gsem1, gsem2, ssem0, ssem1, ssem2) = refs
        else:
            (g_hbm, src_hbm, dst_hbm, out_hbm,
             src_v, dst_v, buf0, buf1, buf2, tab_sh, acc_sh,
             gsem0, gsem1, gsem2, ssem0, ssem1, ssem2) = refs
        bufs = (buf0, buf1, buf2)
        gsem = (gsem0, gsem1, gsem2)
        ssem = (ssem0, ssem1, ssem2)
        cid = lax.axis_index("c")
        sid = lax.axis_index("s")
        r0 = sid * ROWS_PER_SUB

        def _zrow(i, _):
            def _zf(f, __):
                bufs[0][i, pl.ds(f * 16, 16)] = jnp.zeros((16,), _f32)
                return 0
            lax.fori_loop(0, wh // 16, _zf, 0)
            return 0
        lax.fori_loop(0, CHUNK, _zrow, 0)
        for k in range(ROWS_PER_SUB // CHUNK):
            pltpu.sync_copy(bufs[0], acc_sh.at[pl.ds(r0 + k * CHUNK, CHUNK), :])
        pltpu.sync_copy(g_hbm.at[cid, pl.ds(r0, ROWS_PER_SUB), :],
                        tab_sh.at[pl.ds(r0, ROWS_PER_SUB), :])
        plsc.subcore_barrier()

        def _gwait(b):
            pltpu.make_async_copy(tab_sh.at[src_v.at[0]], bufs[b], gsem[b]).wait()

        def _swait(b):
            pltpu.make_async_copy(bufs[b], acc_sh.at[dst_v.at[0]], ssem[b]).wait()

        for h in range(2):
            pltpu.sync_copy(src_hbm.at[sid, pl.ds(h * HALF, HALF)], src_v)
            pltpu.sync_copy(dst_hbm.at[sid, pl.ds(h * HALF, HALF)], dst_v)
            pltpu.async_copy(tab_sh.at[src_v.at[0]], bufs[0], gsem[0])
            pltpu.async_copy(tab_sh.at[src_v.at[1]], bufs[1], gsem[1])
            _gwait(0)
            pltpu.async_copy(bufs[0], acc_sh.at[dst_v.at[0]], ssem[0], add=True)
            pltpu.async_copy(tab_sh.at[src_v.at[2]], bufs[2], gsem[2])
            _gwait(1)
            pltpu.async_copy(bufs[1], acc_sh.at[dst_v.at[1]], ssem[1], add=True)
            _swait(0)
            pltpu.async_copy(tab_sh.at[src_v.at[3]], bufs[0], gsem[0])

            def _step(s, _):
                for u in range(3):
                    j = s * 3 + u + 2
                    b = (u + 2) % 3
                    _gwait(b)
                    pltpu.async_copy(bufs[b], acc_sh.at[dst_v.at[j]], ssem[b], add=True)
                    _swait((b + 2) % 3)
                    pltpu.async_copy(tab_sh.at[src_v.at[j + 2]], bufs[(b + 2) % 3],
                                     gsem[(b + 2) % 3])
                return 0
            lax.fori_loop(0, (HALF - 4) // 3, _step, 0)

            j = HALF - 3
            b = j % 3
            _gwait(b)
            pltpu.async_copy(bufs[b], acc_sh.at[dst_v.at[j]], ssem[b], add=True)
            _swait((b + 2) % 3)
            pltpu.async_copy(tab_sh.at[src_v.at[j + 2]], bufs[(b + 2) % 3],
                             gsem[(b + 2) % 3])
            for j in range(HALF - 2, HALF):
                b = j % 3
                _gwait(b)
                pltpu.async_copy(bufs[b], acc_sh.at[dst_v.at[j]], ssem[b], add=True)
                _swait((b + 2) % 3)
            _swait((HALF - 1) % 3)

        plsc.subcore_barrier()
        if not final:
            pltpu.sync_copy(acc_sh.at[pl.ds(r0, ROWS_PER_SUB), :],
                            out_hbm.at[cid, pl.ds(r0, ROWS_PER_SUB), :])
        else:
            pltpu.sync_copy(dinv_hbm.at[pl.ds(r0, ROWS_PER_SUB)], dinv_v)
            pltpu.sync_copy(b2_hbm.at[cid], b2_v)
            bias = [b2_v[pl.ds(f * 16, 16)] for f in range(wh // 16)]
            for k in range(ROWS_PER_SUB // CHUNK):
                pltpu.sync_copy(acc_sh.at[pl.ds(r0 + k * CHUNK, CHUNK), :], buf0)
                pltpu.sync_copy(tab_sh.at[pl.ds(r0 + k * CHUNK, CHUNK), :], buf1)

                def _row(i, _, k=k):
                    dv = plsc.load_gather(
                        dinv_v, [jnp.broadcast_to(k * CHUNK + i, (16,))])
                    for f in range(wh // 16):
                        cs = pl.ds(f * 16, 16)
                        buf2[i, cs] = (buf0[i, cs] + buf1[i, cs]) * dv + bias[f]
                    return 0
                lax.fori_loop(0, CHUNK, _row, 0)

                rowg = r0 + k * CHUNK

                @pl.when(jnp.logical_and(rowg + CHUNK <= N, cid == 0))
                def _():
                    pltpu.sync_copy(buf2, out_hbm.at[pl.ds(rowg, CHUNK), pl.ds(0, WH2)])

                @pl.when(jnp.logical_and(rowg + CHUNK <= N, cid == 1))
                def _():
                    pltpu.sync_copy(buf2.at[:, pl.ds(0, C - WH2)],
                                    out_hbm.at[pl.ds(rowg, CHUNK), pl.ds(WH2, C - WH2)])

                @pl.when(jnp.logical_and(rowg == (N // CHUNK) * CHUNK, cid == 0))
                def _():
                    pltpu.sync_copy(buf2.at[pl.ds(0, N % CHUNK), :],
                                    out_hbm.at[pl.ds(rowg, N % CHUNK), pl.ds(0, WH2)])

                @pl.when(jnp.logical_and(rowg == (N // CHUNK) * CHUNK, cid == 1))
                def _():
                    pltpu.sync_copy(buf2.at[pl.ds(0, N % CHUNK), pl.ds(0, C - WH2)],
                                    out_hbm.at[pl.ds(rowg, N % CHUNK), pl.ds(WH2, C - WH2)])

    if final:
        out_type = jax.ShapeDtypeStruct((N, C), _f32)
        extra_scr = [pltpu.VMEM((ROWS_PER_SUB,), _f32), pltpu.VMEM((wh,), _f32)]
    else:
        out_type = jax.ShapeDtypeStruct((NCORE, NP, wh), _f32)
        extra_scr = []
    return functools.partial(
        pl.kernel,
        out_type=out_type,
        mesh=plsc.VectorSubcoreMesh(core_axis_name="c", subcore_axis_name="s"),
        compiler_params=pltpu.CompilerParams(
            needs_layout_passes=False,
            use_tc_tiling_on_sc=False,
        ),
        scratch_types=(
            [pltpu.VMEM((HALF, CHUNK), _i32)] * 2
            + [pltpu.VMEM((CHUNK, wh), _f32)] * 3
            + extra_scr
            + [pltpu.VMEM_SHARED((NP, wh), _f32)] * 2
            + [pltpu.SemaphoreType.DMA] * 6
        ),
    )(_body)


_agg_d = _make_agg(WH1)
_agg_w = _make_agg(WH2, final=True)


def _mid_body(agg_ref, g_ref, dinv_ref, w1_ref, b1_ref, w2_ref, q_ref):
    s = jnp.concatenate([agg_ref[0], agg_ref[1]], axis=-1)
    gg = jnp.concatenate([g_ref[0], g_ref[1]], axis=-1)
    dinv = dinv_ref[...]
    agg1 = dinv * (s + gg)
    h1 = jnp.maximum(agg1 @ w1_ref[...] + b1_ref[...], 0.0)
    q = dinv * (h1 @ w2_ref[...])
    q_ref[0, :, :] = q[:, 0:WH2]
    q_ref[1, :, :] = q[:, WH2:2 * WH2]


def _mid_call(agg, g, dinv, W1, b1_2d, W2p):
    return pl.pallas_call(
        _mid_body,
        out_shape=jax.ShapeDtypeStruct((NCORE, NP, WH2), _f32),
    )(agg, g, dinv, W1, b1_2d, W2p)


def kernel(x, edge_index, community, W_lin, b_lin, W1, b1, W2, b2):
    src = edge_index[0]
    dst = edge_index[1]
    pad = EPAD - E
    src3 = jnp.concatenate([src, jnp.zeros((pad,), _i32)]).reshape(NSUB, NCHS, CHUNK)
    dst3 = jnp.concatenate([dst, jnp.full((pad,), DUMP_ROW, _i32)]).reshape(NSUB, NCHS, CHUNK)
    W2p = jnp.pad(W2, ((0, 0), (0, 2 * WH2 - C)))
    b2p = jnp.stack([b2[0:WH2], jnp.pad(b2[WH2:C], (0, 2 * WH2 - C))])

    degp = _deg_call(dst3)
    dinv, dinv1, g = _prep_call(degp, x, community.reshape(N, 1), W_lin,
                                b_lin.reshape(1, H))
    aggp = _agg_d(g, src3, dst3)
    q = _mid_call(aggp, g, dinv, W1, b1.reshape(1, H), W2p)
    return _agg_w(q, src3, dst3, dinv1, b2p)
